# baseline (speedup 1.0000x reference)
import functools
import math

import jax
import jax.numpy as jnp
from jax import lax
from jax.experimental import pallas as pl
from jax.experimental.pallas import tpu as pltpu
from jax.experimental.pallas import tpu_sc as plsc

D_MODEL = 64
SCALE = math.sqrt(D_MODEL)

NUM_CORES = 2
NUM_SUBCORES = 16
NW = NUM_CORES * NUM_SUBCORES
LANES = 16


def _worker_id():
    return lax.axis_index("s") * NUM_CORES + lax.axis_index("c")


def _make_transpose(V: int):
    n_blocks = V // 128
    n_tail = V - n_blocks * 128
    k_max = -(-n_blocks // NW)
    k_iters = (k_max + 1) // 2

    mesh = plsc.VectorSubcoreMesh(core_axis_name="c", subcore_axis_name="s")

    @functools.partial(
        pl.kernel,
        mesh=mesh,
        compiler_params=pltpu.CompilerParams(needs_layout_passes=False),
        out_type=jax.ShapeDtypeStruct((V // 2, 128), jnp.float32),
        scratch_types=[
            pltpu.VMEM((2, 64, 128), jnp.float32),
            pltpu.VMEM((2, 64, 128), jnp.float32),
            pltpu.VMEM((n_tail // 2, 128), jnp.float32),
            pltpu.SemaphoreType.DMA,
            pltpu.SemaphoreType.DMA,
        ],
    )
    def transpose_pack(wt_hbm, wtail_hbm, wp_hbm, wt_v, pt_v, tail_v, lsem, ssem):
        wid = _worker_id()

        def blk(k):
            return k * NW + wid

        def start_load(k, b):
            @pl.when(blk(k) < n_blocks)
            def _():
                pltpu.async_copy(
                    wt_hbm.at[:, pl.ds(blk(k) * 128, 128)], wt_v.at[b], lsem
                )

        def wait_load(k, b):
            @pl.when(blk(k) < n_blocks)
            def _():
                pltpu.make_async_copy(
                    wt_hbm.at[:, pl.ds(0, 128)], wt_v.at[b], lsem
                ).wait()

        def start_store(k, b):
            @pl.when(blk(k) < n_blocks)
            def _():
                pltpu.async_copy(
                    pt_v.at[b], wp_hbm.at[pl.ds(blk(k) * 64, 64)], ssem
                )

        def wait_store(k, b):
            @pl.when((k >= 0) & (blk(k) < n_blocks))
            def _():
                pltpu.make_async_copy(
                    pt_v.at[b], wp_hbm.at[pl.ds(0, 64)], ssem
                ).wait()

        def transpose_block(k, b):
            @pl.when(blk(k) < n_blocks)
            def _():
                iotas = [lax.iota(jnp.int32, LANES) + g * LANES for g in range(4)]

                def row_body(j, carry):
                    for p in range(2):
                        col = jnp.full((LANES,), 2 * j + p, jnp.int32)
                        for g in range(4):
                            vals = plsc.load_gather(wt_v.at[b], [iotas[g], col])
                            pt_v[b, j, pl.ds((p * 4 + g) * LANES, LANES)] = vals
                    return carry

                lax.fori_loop(0, 64, row_body, 0, unroll=2)

        @pl.when(wid == 0)
        def _():
            pltpu.sync_copy(wtail_hbm, tail_v)
            pltpu.sync_copy(tail_v, wp_hbm.at[pl.ds(n_blocks * 64, n_tail // 2)])

        start_load(0, 0)

        def outer(kk, carry):
            for b in range(2):
                k = kk * 2 + b
                wait_load(k, b)
                start_load(k + 1, 1 - b)
                wait_store(k - 2, b)
                transpose_block(k, b)
                start_store(k, b)
            return carry

        lax.fori_loop(0, k_iters, outer, 0)
        wait_store(k_max - 1, (k_max - 1) % 2)
        wait_store(k_max, k_max % 2)

    return transpose_pack


def _make_gather(B0: int, F: int):
    n_btiles = B0 // 128
    t_per_w = n_btiles // NW
    FO = F * 8

    mesh = plsc.VectorSubcoreMesh(core_axis_name="c", subcore_axis_name="s")

    @functools.partial(
        pl.kernel,
        mesh=mesh,
        compiler_params=pltpu.CompilerParams(needs_layout_passes=False),
        out_type=jax.ShapeDtypeStruct((FO, n_btiles, 8, 128), jnp.float32),
        scratch_types=[
            pltpu.VMEM((F, 128), jnp.int32),
            pltpu.VMEM((F, 128), jnp.int32),
            pltpu.VMEM((F, 128), jnp.int32),
            pltpu.VMEM((2, 128, 128), jnp.float32),
            pltpu.VMEM((2, 8, 8, 128), jnp.float32),
            pltpu.SemaphoreType.DMA,
            pltpu.SemaphoreType.DMA,
        ],
    )
    def gather_scale(xt_hbm, wp_hbm, out_hbm, x_v, pidx_v, poff_v, g_v, s_v,
                     gsem, ssem):
        wid = _worker_id()

        def start_gather(f, b):
            pltpu.async_copy(wp_hbm.at[pidx_v.at[f]], g_v.at[b], gsem)

        def wait_gather(f, b):
            pltpu.make_async_copy(wp_hbm.at[pidx_v.at[f]], g_v.at[b], gsem).wait()

        def start_store(f, t, b):
            pltpu.async_copy(s_v.at[b], out_hbm.at[pl.ds(f * 8, 8), t], ssem)

        def wait_store(b):
            pltpu.make_async_copy(
                s_v.at[b], out_hbm.at[pl.ds(0, 8), 0], ssem
            ).wait()

        def t_body(ti, carry):
            t = wid * t_per_w + ti
            pltpu.sync_copy(xt_hbm.at[:, pl.ds(t * 128, 128)], x_v)

            def idx_body(f, c):
                for lg in range(8):
                    sl = pl.ds(lg * LANES, LANES)
                    xv = x_v[f, sl]
                    pidx_v[f, sl] = lax.shift_right_logical(xv, 1)
                    poff_v[f, sl] = (xv & 1) * D_MODEL
                return c

            lax.fori_loop(0, F, idx_body, 0, unroll=2)

            start_gather(0, 0)

            def f_body(ff, c):
                for b in range(2):
                    f = ff * 2 + b
                    wait_gather(f, b)

                    @pl.when(f + 1 < F)
                    def _():
                        start_gather(f + 1, 1 - b)

                    @pl.when((f >= 2) | (ti > 0))
                    def _():
                        wait_store(b)

                    iotas = [lax.iota(jnp.int32, LANES) + g * LANES
                             for g in range(8)]

                    def slab_body(oo, cc):
                        for lg in range(8):
                            sl = pl.ds(lg * LANES, LANES)
                            base = poff_v[f, sl] + oo * 8
                            for s in range(8):
                                vals = plsc.load_gather(
                                    g_v.at[b], [iotas[lg], base + s]
                                )
                                s_v[b, oo, s, sl] = vals * SCALE
                        return cc

                    lax.fori_loop(0, 8, slab_body, 0)
                    start_store(f, t, b)
                return c

            lax.fori_loop(0, F // 2, f_body, 0)
            return carry

        lax.fori_loop(0, t_per_w, t_body, 0)
        wait_store(0)
        wait_store(1)

    return gather_scale


def kernel(x, W):
    B0, F = x.shape
    V, D = W.shape
    xt = x.T.astype(jnp.int32)
    wt = W.T
    n_tail = V % 128
    wtail = W[V - n_tail:].reshape(n_tail // 2, 128)
    wp = _make_transpose(V)(wt, wtail)
    out4 = _make_gather(B0, F)(xt, wp)
    out = (
        out4.reshape(F, 8, B0 // 128, 8, 128)
        .transpose(2, 4, 0, 1, 3)
        .reshape(B0, F, D)
    )
    return out


# speedup vs baseline: 1.8325x; 1.8325x over previous
import functools
import math

import jax
import jax.numpy as jnp
from jax import lax
from jax.experimental import pallas as pl
from jax.experimental.pallas import tpu as pltpu
from jax.experimental.pallas import tpu_sc as plsc

D_MODEL = 64
SCALE = math.sqrt(D_MODEL)

NUM_CORES = 2
NUM_SUBCORES = 16
NW = NUM_CORES * NUM_SUBCORES
LANES = 16


def _worker_id():
    return lax.axis_index("s") * NUM_CORES + lax.axis_index("c")


def _make_transpose(V: int):
    n_blocks = V // 128
    n_tail = V - n_blocks * 128
    k_max = -(-n_blocks // NW)
    k_iters = (k_max + 1) // 2

    mesh = plsc.VectorSubcoreMesh(core_axis_name="c", subcore_axis_name="s")

    @functools.partial(
        pl.kernel,
        mesh=mesh,
        compiler_params=pltpu.CompilerParams(needs_layout_passes=False),
        out_type=jax.ShapeDtypeStruct((V // 2, 128), jnp.float32),
        scratch_types=[
            pltpu.VMEM((2, 64, 128), jnp.float32),
            pltpu.VMEM((2, 64, 128), jnp.float32),
            pltpu.VMEM((n_tail // 2, 128), jnp.float32),
            pltpu.SemaphoreType.DMA,
            pltpu.SemaphoreType.DMA,
        ],
    )
    def transpose_pack(wt_hbm, wtail_hbm, wp_hbm, wt_v, pt_v, tail_v, lsem, ssem):
        wid = _worker_id()

        def blk(k):
            return k * NW + wid

        def start_load(k, b):
            @pl.when(blk(k) < n_blocks)
            def _():
                pltpu.async_copy(
                    wt_hbm.at[:, pl.ds(blk(k) * 128, 128)], wt_v.at[b], lsem
                )

        def wait_load(k, b):
            @pl.when(blk(k) < n_blocks)
            def _():
                pltpu.make_async_copy(
                    wt_hbm.at[:, pl.ds(0, 128)], wt_v.at[b], lsem
                ).wait()

        def start_store(k, b):
            @pl.when(blk(k) < n_blocks)
            def _():
                pltpu.async_copy(
                    pt_v.at[b], wp_hbm.at[pl.ds(blk(k) * 64, 64)], ssem
                )

        def wait_store(k, b):
            @pl.when((k >= 0) & (blk(k) < n_blocks))
            def _():
                pltpu.make_async_copy(
                    pt_v.at[b], wp_hbm.at[pl.ds(0, 64)], ssem
                ).wait()

        def transpose_block(k, b):
            @pl.when(blk(k) < n_blocks)
            def _():
                iotas = [lax.iota(jnp.int32, LANES) + g * LANES for g in range(4)]

                @plsc.parallel_loop(0, 64, unroll=8)
                def row_body(j):
                    for p in range(2):
                        col = jnp.full((LANES,), 2 * j + p, jnp.int32)
                        for g in range(4):
                            vals = plsc.load_gather(wt_v.at[b], [iotas[g], col])
                            pt_v[b, j, pl.ds((p * 4 + g) * LANES, LANES)] = vals

        @pl.when(wid == 0)
        def _():
            pltpu.sync_copy(wtail_hbm, tail_v)
            pltpu.sync_copy(tail_v, wp_hbm.at[pl.ds(n_blocks * 64, n_tail // 2)])

        start_load(0, 0)

        def outer(kk, carry):
            for b in range(2):
                k = kk * 2 + b
                wait_load(k, b)
                start_load(k + 1, 1 - b)
                wait_store(k - 2, b)
                transpose_block(k, b)
                start_store(k, b)
            return carry

        lax.fori_loop(0, k_iters, outer, 0)
        wait_store(k_max - 1, (k_max - 1) % 2)
        wait_store(k_max, k_max % 2)

    return transpose_pack


def _make_gather(B0: int, F: int):
    n_btiles = B0 // 128
    t_per_w = n_btiles // NW
    FO = F * 8

    mesh = plsc.VectorSubcoreMesh(core_axis_name="c", subcore_axis_name="s")

    @functools.partial(
        pl.kernel,
        mesh=mesh,
        compiler_params=pltpu.CompilerParams(needs_layout_passes=False),
        out_type=jax.ShapeDtypeStruct((FO, n_btiles, 8, 128), jnp.float32),
        scratch_types=[
            pltpu.VMEM((F, 128), jnp.int32),
            pltpu.VMEM((F, 128), jnp.int32),
            pltpu.VMEM((F, 128), jnp.int32),
            pltpu.VMEM((2, 128, 128), jnp.float32),
            pltpu.VMEM((2, 8, 8, 128), jnp.float32),
            pltpu.SemaphoreType.DMA,
            pltpu.SemaphoreType.DMA,
        ],
    )
    def gather_scale(xt_hbm, wp_hbm, out_hbm, x_v, pidx_v, poff_v, g_v, s_v,
                     gsem, ssem):
        wid = _worker_id()

        def start_gather(f, b):
            pltpu.async_copy(wp_hbm.at[pidx_v.at[f]], g_v.at[b], gsem)

        def wait_gather(f, b):
            pltpu.make_async_copy(wp_hbm.at[pidx_v.at[f]], g_v.at[b], gsem).wait()

        def start_store(f, t, b):
            pltpu.async_copy(s_v.at[b], out_hbm.at[pl.ds(f * 8, 8), t], ssem)

        def wait_store(b):
            pltpu.make_async_copy(
                s_v.at[b], out_hbm.at[pl.ds(0, 8), 0], ssem
            ).wait()

        def t_body(ti, carry):
            t = wid * t_per_w + ti
            pltpu.sync_copy(xt_hbm.at[:, pl.ds(t * 128, 128)], x_v)

            def idx_body(f, c):
                for lg in range(8):
                    sl = pl.ds(lg * LANES, LANES)
                    xv = x_v[f, sl]
                    pidx_v[f, sl] = lax.shift_right_logical(xv, 1)
                    poff_v[f, sl] = (xv & 1) * D_MODEL
                return c

            lax.fori_loop(0, F, idx_body, 0, unroll=2)

            start_gather(0, 0)

            def f_body(ff, c):
                for b in range(2):
                    f = ff * 2 + b
                    wait_gather(f, b)

                    @pl.when(f + 1 < F)
                    def _():
                        start_gather(f + 1, 1 - b)

                    @pl.when((f >= 2) | (ti > 0))
                    def _():
                        wait_store(b)

                    iotas = [lax.iota(jnp.int32, LANES) + g * LANES
                             for g in range(8)]
                    pv = [poff_v[f, pl.ds(lg * LANES, LANES)] for lg in range(8)]

                    @plsc.parallel_loop(0, 8, unroll=4)
                    def slab_body(oo):
                        for lg in range(8):
                            base = pv[lg] + oo * 8
                            for s in range(8):
                                vals = plsc.load_gather(
                                    g_v.at[b], [iotas[lg], base + s]
                                )
                                s_v[b, oo, s, pl.ds(lg * LANES, LANES)] = vals * SCALE
                    start_store(f, t, b)
                return c

            lax.fori_loop(0, F // 2, f_body, 0)
            return carry

        lax.fori_loop(0, t_per_w, t_body, 0)
        wait_store(0)
        wait_store(1)

    return gather_scale


def kernel(x, W):
    B0, F = x.shape
    V, D = W.shape
    xt = x.T.astype(jnp.int32)
    wt = W.T
    n_tail = V % 128
    wtail = W[V - n_tail:].reshape(n_tail // 2, 128)
    wp = _make_transpose(V)(wt, wtail)
    out4 = _make_gather(B0, F)(xt, wp)
    out = (
        out4.reshape(F, 8, B0 // 128, 8, 128)
        .transpose(2, 4, 0, 1, 3)
        .reshape(B0, F, D)
    )
    return out


# speedup vs baseline: 2.1882x; 1.1941x over previous
import functools
import math

import jax
import jax.numpy as jnp
from jax import lax
from jax.experimental import pallas as pl
from jax.experimental.pallas import tpu as pltpu
from jax.experimental.pallas import tpu_sc as plsc

D_MODEL = 64
SCALE = math.sqrt(D_MODEL)

NUM_CORES = 2
NUM_SUBCORES = 16
NW = NUM_CORES * NUM_SUBCORES
LANES = 16


def _worker_id():
    return lax.axis_index("s") * NUM_CORES + lax.axis_index("c")


def _make_transpose(V: int):
    n_blocks = V // 128
    n_tail = V - n_blocks * 128
    k_max = -(-n_blocks // NW)
    k_iters = (k_max + 1) // 2

    mesh = plsc.VectorSubcoreMesh(core_axis_name="c", subcore_axis_name="s")

    @functools.partial(
        pl.kernel,
        mesh=mesh,
        compiler_params=pltpu.CompilerParams(needs_layout_passes=False),
        out_type=jax.ShapeDtypeStruct((V // 2, 128), jnp.float32),
        scratch_types=[
            pltpu.VMEM((2, 64, 128), jnp.float32),
            pltpu.VMEM((2, 64, 128), jnp.float32),
            pltpu.VMEM((n_tail // 2, 128), jnp.float32),
            pltpu.SemaphoreType.DMA,
            pltpu.SemaphoreType.DMA,
        ],
    )
    def transpose_pack(wt_hbm, wtail_hbm, wp_hbm, wt_v, pt_v, tail_v, lsem, ssem):
        wid = _worker_id()

        def blk(k):
            return k * NW + wid

        def start_load(k, b):
            @pl.when(blk(k) < n_blocks)
            def _():
                pltpu.async_copy(
                    wt_hbm.at[:, pl.ds(blk(k) * 128, 128)], wt_v.at[b], lsem
                )

        def wait_load(k, b):
            @pl.when(blk(k) < n_blocks)
            def _():
                pltpu.make_async_copy(
                    wt_hbm.at[:, pl.ds(0, 128)], wt_v.at[b], lsem
                ).wait()

        def start_store(k, b):
            @pl.when(blk(k) < n_blocks)
            def _():
                pltpu.async_copy(
                    pt_v.at[b], wp_hbm.at[pl.ds(blk(k) * 64, 64)], ssem
                )

        def wait_store(k, b):
            @pl.when((k >= 0) & (blk(k) < n_blocks))
            def _():
                pltpu.make_async_copy(
                    pt_v.at[b], wp_hbm.at[pl.ds(0, 64)], ssem
                ).wait()

        def transpose_block(k, b):
            @pl.when(blk(k) < n_blocks)
            def _():
                iotas = [lax.iota(jnp.int32, LANES) + g * LANES for g in range(4)]

                @plsc.parallel_loop(0, 64, unroll=8)
                def row_body(j):
                    for p in range(2):
                        col = jnp.full((LANES,), 2 * j + p, jnp.int32)
                        for g in range(4):
                            vals = plsc.load_gather(wt_v.at[b], [iotas[g], col])
                            pt_v[b, j, pl.ds((p * 4 + g) * LANES, LANES)] = vals

        @pl.when(wid == 0)
        def _():
            pltpu.sync_copy(wtail_hbm, tail_v)
            pltpu.sync_copy(tail_v, wp_hbm.at[pl.ds(n_blocks * 64, n_tail // 2)])

        start_load(0, 0)

        def outer(kk, carry):
            for b in range(2):
                k = kk * 2 + b
                wait_load(k, b)
                start_load(k + 1, 1 - b)
                wait_store(k - 2, b)
                transpose_block(k, b)
                start_store(k, b)
            return carry

        lax.fori_loop(0, k_iters, outer, 0)
        wait_store(k_max - 1, (k_max - 1) % 2)
        wait_store(k_max, k_max % 2)

    return transpose_pack


def _make_gather(B0: int, F: int):
    n_btiles = B0 // 128
    t_per_w = n_btiles // NW
    FO = F * 8

    mesh = plsc.VectorSubcoreMesh(core_axis_name="c", subcore_axis_name="s")

    @functools.partial(
        pl.kernel,
        mesh=mesh,
        compiler_params=pltpu.CompilerParams(needs_layout_passes=False),
        out_type=jax.ShapeDtypeStruct((FO, n_btiles, 8, 128), jnp.float32),
        scratch_types=[
            pltpu.VMEM((F, 128), jnp.int32),
            pltpu.VMEM((F, 128), jnp.int32),
            pltpu.VMEM((F, 128), jnp.int32),
            pltpu.VMEM((2, 128, 128), jnp.float32),
            pltpu.VMEM((2, 8, 8, 128), jnp.float32),
            pltpu.SemaphoreType.DMA,
            pltpu.SemaphoreType.DMA,
        ],
    )
    def gather_scale(xt_hbm, wp_hbm, out_hbm, x_v, pidx_v, poff_v, g_v, s_v,
                     gsem, ssem):
        wid = _worker_id()

        def start_gather(f, b):
            pltpu.async_copy(wp_hbm.at[pidx_v.at[f]], g_v.at[b], gsem)

        def wait_gather(f, b):
            pltpu.make_async_copy(wp_hbm.at[pidx_v.at[f]], g_v.at[b], gsem).wait()

        def start_store(f, t, b):
            pltpu.async_copy(s_v.at[b], out_hbm.at[pl.ds(f * 8, 8), t], ssem)

        def wait_store(b):
            pltpu.make_async_copy(
                s_v.at[b], out_hbm.at[pl.ds(0, 8), 0], ssem
            ).wait()

        def t_body(ti, carry):
            t = wid * t_per_w + ti
            pltpu.sync_copy(xt_hbm.at[:, pl.ds(t * 128, 128)], x_v)

            def idx_body(f, c):
                for lg in range(8):
                    sl = pl.ds(lg * LANES, LANES)
                    xv = x_v[f, sl]
                    pidx_v[f, sl] = lax.shift_right_logical(xv, 1)
                    poff_v[f, sl] = (xv & 1) * D_MODEL
                return c

            lax.fori_loop(0, F, idx_body, 0, unroll=2)

            start_gather(0, 0)

            def f_body(ff, c):
                for b in range(2):
                    f = ff * 2 + b
                    wait_gather(f, b)

                    @pl.when(f + 1 < F)
                    def _():
                        start_gather(f + 1, 1 - b)

                    @pl.when((f >= 2) | (ti > 0))
                    def _():
                        wait_store(b)

                    iotas = [lax.iota(jnp.int32, LANES) + g * LANES
                             for g in range(8)]
                    pv = [poff_v[f, pl.ds(lg * LANES, LANES)] for lg in range(8)]

                    @plsc.parallel_loop(0, 8, unroll=4)
                    def slab_body(oo):
                        for lg in range(8):
                            base = pv[lg] + oo * 8
                            for s in range(8):
                                vals = plsc.load_gather(
                                    g_v.at[b], [iotas[lg], base + s]
                                )
                                s_v[b, oo, s, pl.ds(lg * LANES, LANES)] = vals * SCALE
                    start_store(f, t, b)
                return c

            lax.fori_loop(0, F // 2, f_body, 0)
            return carry

        lax.fori_loop(0, t_per_w, t_body, 0)
        wait_store(0)
        wait_store(1)

    return gather_scale


def kernel(x, W):
    B0, F = x.shape
    V, D = W.shape
    xt = x.T.astype(jnp.int32)
    wp = W.reshape(V // 2, 2 * D)
    out4 = _make_gather(B0, F)(xt, wp)
    out = (
        out4.reshape(F, 8, B0 // 128, 8, 128)
        .transpose(2, 4, 0, 1, 3)
        .reshape(B0, F, D)
    )
    return out
